# baseline (speedup 1.0000x reference)
import functools

import jax
import jax.numpy as jnp
from jax import lax
from jax.experimental import pallas as pl
from jax.experimental.pallas import tpu as pltpu
from jax.experimental.pallas import tpu_sc as plsc

NUM_CAT = 26
NUM_SCALAR = 16
VOCAB = 100000
ROWS = VOCAB + 1
EMB = 32
FEAT = NUM_CAT + NUM_SCALAR

B0 = 1024
B1 = 50
N = B0 * B1

NC = 2
NS = 16
NW = NC * NS
TW = N // NW

VB1 = 512
SPT = 196
NSLAB = NUM_CAT * SPT
TAIL0 = 195 * VB1
PV = SPT * VB1
SLABW = EMB * VB1
NPW = NUM_CAT * PV * EMB

T = 64
R = NUM_CAT * T
GB = 128
NG = R // GB
CHUNKS = TW // T
S16 = R // 16


_mesh = plsc.VectorSubcoreMesh(core_axis_name="c", subcore_axis_name="s")
_cpt = pltpu.CompilerParams(use_tc_tiling_on_sc=True, needs_layout_passes=False)


@functools.partial(
    pl.kernel,
    out_type=jax.ShapeDtypeStruct((NPW,), jnp.float32),
    mesh=_mesh,
    compiler_params=_cpt,
    scratch_types=[
        pltpu.VMEM((EMB, VB1), jnp.float32),
        pltpu.VMEM((EMB, VB1), jnp.float32),
        pltpu.VMEM((SLABW,), jnp.float32),
        pltpu.VMEM((SLABW,), jnp.float32),
        pltpu.SemaphoreType.DMA,
        pltpu.SemaphoreType.DMA,
    ],
)
def _pack_sc(tt_hbm, tail_hbm, p_hbm, sbuf0, sbuf1, pbuf0, pbuf1, isem, osem):
    wid = lax.axis_index("s") * NC + lax.axis_index("c")
    j = lax.iota(jnp.int32, 16)
    cs = ((j >> 2) << 7) + ((j & 3) << 5)

    def stage_in(sid, sbuf):
        i = sid // SPT
        vb = sid - i * SPT

        @pl.when(vb < SPT - 1)
        def _():
            pltpu.async_copy(tt_hbm.at[i, :, pl.ds(vb * VB1, VB1)], sbuf, isem)

        @pl.when(vb == SPT - 1)
        def _():
            pltpu.async_copy(tail_hbm.at[i], sbuf, isem)

    fvs = [(j + d) & 15 for d in range(16)]

    def transpose_slab(sbuf, pbuf):
        def blk_body(blk, _):
            h = blk >> 5
            mb = blk & 31
            f0 = h << 4
            vv = j + (mb << 4)
            vvt = (vv << 5) + f0
            for d in range(16):
                fvf = fvs[d] + f0
                xv = plsc.load_gather(sbuf, [fvf, vv])
                plsc.store_scatter(pbuf, [vvt + fvs[d]], xv)
            return 0

        lax.fori_loop(0, 64, blk_body, 0)

    def wait_in(sbuf):
        pltpu.make_async_copy(tail_hbm.at[0], sbuf, isem).wait()

    def start_out(sid, pbuf):
        i = sid // SPT
        vb = sid - i * SPT
        pltpu.async_copy(
            pbuf, p_hbm.at[pl.ds(i * (PV * EMB) + vb * SLABW, SLABW)], osem
        )

    def wait_out(pbuf):
        pltpu.make_async_copy(pbuf, p_hbm.at[pl.ds(0, SLABW)], osem).wait()

    def pair_body(kk, _):
        s0 = wid + (2 * kk) * NW
        s1 = wid + (2 * kk + 1) * NW
        g0 = s0 < NSLAB
        g1 = s1 < NSLAB

        @pl.when(g0)
        def _():
            stage_in(s0, sbuf0)

        @pl.when(g1)
        def _():
            stage_in(s1, sbuf1)

        @pl.when(g0)
        def _():
            wait_in(sbuf0)
            transpose_slab(sbuf0, pbuf0)
            start_out(s0, pbuf0)

        @pl.when(g1)
        def _():
            wait_in(sbuf1)
            transpose_slab(sbuf1, pbuf1)
            start_out(s1, pbuf1)

        @pl.when(g0)
        def _():
            wait_out(pbuf0)

        @pl.when(g1)
        def _():
            wait_out(pbuf1)

        return 0

    nsl = (NSLAB + NW - 1) // NW
    lax.fori_loop(0, nsl // 2, pair_body, 0)


def _proj_body(x_ref, wt_ref, b_ref, o_ref):
    xs = x_ref[:, NUM_CAT:FEAT]
    o_ref[...] = (
        jnp.dot(xs, wt_ref[...], preferred_element_type=jnp.float32) + b_ref[...]
    )


_BT = 2048


def _proj_tc(xf, wt, b2):
    return pl.pallas_call(
        _proj_body,
        grid=(N // _BT,),
        in_specs=[
            pl.BlockSpec((_BT, FEAT), lambda i: (i, 0)),
            pl.BlockSpec((NUM_SCALAR, EMB), lambda i: (0, 0)),
            pl.BlockSpec((1, EMB), lambda i: (0, 0)),
        ],
        out_specs=pl.BlockSpec((_BT, EMB), lambda i: (i, 0)),
        out_shape=jax.ShapeDtypeStruct((N, EMB), jnp.float32),
    )(xf, wt, b2)


_cp = pltpu.CompilerParams(use_tc_tiling_on_sc=False, needs_layout_passes=False)


@functools.partial(
    pl.kernel,
    out_type=jax.ShapeDtypeStruct((N, EMB), jnp.float32),
    mesh=_mesh,
    compiler_params=_cp,
    scratch_types=[
        pltpu.VMEM((T * FEAT,), jnp.float32),
        pltpu.VMEM((R,), jnp.int32),
        pltpu.VMEM((R, EMB), jnp.float32),
        pltpu.VMEM((T, EMB), jnp.float32),
        pltpu.SemaphoreType.DMA,
    ],
)
def _sc_lookup(xflat_hbm, tab_hbm, proj_hbm, out_hbm, xbuf, gidx, gbuf, obuf, gsem):
    wid = lax.axis_index("s") * NC + lax.axis_index("c")
    wbase = wid * TW

    def chunk_body(c, _):
        tok0 = wbase + c * T

        pltpu.sync_copy(xflat_hbm.at[pl.ds(tok0 * FEAT, T * FEAT)], xbuf)

        def idx_body(s, _):
            r0 = s * 16
            rv = lax.iota(jnp.int32, 16) + r0
            q = rv // NUM_CAT
            cat = rv - q * NUM_CAT
            xv = plsc.load_gather(xbuf, [q * FEAT + cat])
            gidx[pl.ds(r0, 16)] = xv.astype(jnp.int32) + cat * PV
            return 0

        lax.fori_loop(0, S16, idx_body, 0, unroll=4)

        handles = [
            pltpu.async_copy(
                tab_hbm.at[gidx.at[pl.ds(j * GB, GB)]],
                gbuf.at[pl.ds(j * GB, GB)],
                gsem,
            )
            for j in range(NG)
        ]

        pltpu.sync_copy(proj_hbm.at[pl.ds(tok0, T)], obuf)

        for h in handles:
            h.wait()

        def acc_body(t, _):
            r0 = t * NUM_CAT
            v0 = obuf[t, pl.ds(0, 16)]
            v1 = obuf[t, pl.ds(16, 16)]
            for i in range(NUM_CAT):
                v0 = v0 + gbuf[r0 + i, pl.ds(0, 16)]
                v1 = v1 + gbuf[r0 + i, pl.ds(16, 16)]
            obuf[t, pl.ds(0, 16)] = v0
            obuf[t, pl.ds(16, 16)] = v1
            return 0

        lax.fori_loop(0, T, acc_body, 0)

        pltpu.sync_copy(obuf, out_hbm.at[pl.ds(tok0, T)])
        return 0

    lax.fori_loop(0, CHUNKS, chunk_body, 0)


def kernel(x, tables, W, b):
    tt = jnp.transpose(tables, (0, 2, 1))
    tail = jnp.pad(tt[:, :, TAIL0:], ((0, 0), (0, 0), (0, TAIL0 + VB1 - ROWS)))
    packed = _pack_sc(tt, tail)
    tabf = packed.reshape(NPW // EMB, EMB)
    xf = x.reshape(N, FEAT)
    proj = _proj_tc(xf, W.T, b.reshape(1, EMB))
    out = _sc_lookup(xf.reshape(N * FEAT), tabf, proj)
    return out.reshape(B0, B1, EMB)


# speedup vs baseline: 1.3103x; 1.3103x over previous
import functools

import jax
import jax.numpy as jnp
from jax import lax
from jax.experimental import pallas as pl
from jax.experimental.pallas import tpu as pltpu
from jax.experimental.pallas import tpu_sc as plsc

NUM_CAT = 26
NUM_SCALAR = 16
VOCAB = 100000
ROWS = VOCAB + 1
EMB = 32
FEAT = NUM_CAT + NUM_SCALAR

B0 = 1024
B1 = 50
N = B0 * B1

NC = 2
NS = 16
NW = NC * NS
TW = N // NW

VB1 = 512
SPT = 196
NSLAB = NUM_CAT * SPT
TAIL0 = 195 * VB1
PV = SPT * VB1
SLABW16 = VB1 * 16
NPW16 = NUM_CAT * PV * 16

T = 64
R = NUM_CAT * T
GB = 128
NG = R // GB
CHUNKS = TW // T
S16 = R // 16


_mesh = plsc.VectorSubcoreMesh(core_axis_name="c", subcore_axis_name="s")
_cpt = pltpu.CompilerParams(use_tc_tiling_on_sc=True, needs_layout_passes=False)


@functools.partial(
    pl.kernel,
    out_type=jax.ShapeDtypeStruct((NPW16,), jnp.float32),
    mesh=_mesh,
    compiler_params=_cpt,
    scratch_types=[
        pltpu.VMEM((EMB, VB1), jnp.float32),
        pltpu.VMEM((EMB, VB1), jnp.float32),
        pltpu.VMEM((SLABW16,), jnp.float32),
        pltpu.VMEM((SLABW16,), jnp.float32),
        pltpu.SemaphoreType.DMA,
        pltpu.SemaphoreType.DMA,
    ],
)
def _pack_sc(tt_hbm, tail_hbm, p_hbm, sbuf0, sbuf1, pbuf0, pbuf1, isem, osem):
    wid = lax.axis_index("s") * NC + lax.axis_index("c")
    j = lax.iota(jnp.int32, 16)

    def stage_in(sid, sbuf):
        i = sid // SPT
        vb = sid - i * SPT

        @pl.when(vb < SPT - 1)
        def _():
            pltpu.async_copy(tt_hbm.at[i, :, pl.ds(vb * VB1, VB1)], sbuf, isem)

        @pl.when(vb == SPT - 1)
        def _():
            pltpu.async_copy(tail_hbm.at[i], sbuf, isem)

    fvs = [(j + d) & 15 for d in range(16)]

    def transpose_slab(sbuf, pbuf):
        def blk_body(mb, _):
            vv = j + (mb << 4)
            vvt = vv << 4
            for d in range(16):
                fa = fvs[d] << 1
                a = plsc.load_gather(sbuf, [fa, vv])
                b2 = plsc.load_gather(sbuf, [fa + 1, vv])
                pk = plsc.pack(a, b2, format=plsc.PackFormat.INTERLEAVED)
                xw = plsc.bitcast(pk, jnp.float32)
                plsc.store_scatter(pbuf, [vvt + fvs[d]], xw)
            return 0

        lax.fori_loop(0, 32, blk_body, 0)

    def wait_in(sbuf):
        pltpu.make_async_copy(tail_hbm.at[0], sbuf, isem).wait()

    def start_out(sid, pbuf):
        i = sid // SPT
        vb = sid - i * SPT
        pltpu.async_copy(
            pbuf, p_hbm.at[pl.ds(i * (PV * 16) + vb * SLABW16, SLABW16)], osem
        )

    def wait_out(pbuf):
        pltpu.make_async_copy(pbuf, p_hbm.at[pl.ds(0, SLABW16)], osem).wait()

    def pair_body(kk, _):
        s0 = wid + (2 * kk) * NW
        s1 = wid + (2 * kk + 1) * NW
        g0 = s0 < NSLAB
        g1 = s1 < NSLAB

        @pl.when(g0)
        def _():
            stage_in(s0, sbuf0)

        @pl.when(g1)
        def _():
            stage_in(s1, sbuf1)

        @pl.when(g0)
        def _():
            wait_in(sbuf0)
            transpose_slab(sbuf0, pbuf0)
            start_out(s0, pbuf0)

        @pl.when(g1)
        def _():
            wait_in(sbuf1)
            transpose_slab(sbuf1, pbuf1)
            start_out(s1, pbuf1)

        @pl.when(g0)
        def _():
            wait_out(pbuf0)

        @pl.when(g1)
        def _():
            wait_out(pbuf1)

        return 0

    nsl = (NSLAB + NW - 1) // NW
    lax.fori_loop(0, nsl // 2, pair_body, 0)


def _proj_body(x_ref, wt_ref, b_ref, o_ref):
    xs = x_ref[:, NUM_CAT:FEAT]
    o_ref[...] = (
        jnp.dot(xs, wt_ref[...], preferred_element_type=jnp.float32) + b_ref[...]
    )


_BT = 2048


def _proj_tc(xf, wt, b2):
    return pl.pallas_call(
        _proj_body,
        grid=(N // _BT,),
        in_specs=[
            pl.BlockSpec((_BT, FEAT), lambda i: (i, 0)),
            pl.BlockSpec((NUM_SCALAR, EMB), lambda i: (0, 0)),
            pl.BlockSpec((1, EMB), lambda i: (0, 0)),
        ],
        out_specs=pl.BlockSpec((_BT, EMB), lambda i: (i, 0)),
        out_shape=jax.ShapeDtypeStruct((N, EMB), jnp.float32),
    )(xf, wt, b2)


_cp = pltpu.CompilerParams(use_tc_tiling_on_sc=False, needs_layout_passes=False)


@functools.partial(
    pl.kernel,
    out_type=jax.ShapeDtypeStruct((N, EMB), jnp.float32),
    mesh=_mesh,
    compiler_params=_cp,
    scratch_types=[
        pltpu.VMEM((T * FEAT,), jnp.float32),
        pltpu.VMEM((R,), jnp.int32),
        pltpu.VMEM((R, 16), jnp.float32),
        pltpu.VMEM((T, EMB), jnp.float32),
        pltpu.SemaphoreType.DMA,
    ],
)
def _sc_lookup(xflat_hbm, tab_hbm, proj_hbm, out_hbm, xbuf, gidx, gbuf, obuf, gsem):
    wid = lax.axis_index("s") * NC + lax.axis_index("c")
    wbase = wid * TW
    ev = lax.iota(jnp.int32, 16) << 1
    od = ev + 1

    def chunk_body(c, _):
        tok0 = wbase + c * T

        pltpu.sync_copy(xflat_hbm.at[pl.ds(tok0 * FEAT, T * FEAT)], xbuf)

        def idx_body(s, _):
            r0 = s * 16
            rv = lax.iota(jnp.int32, 16) + r0
            q = rv // NUM_CAT
            cat = rv - q * NUM_CAT
            xv = plsc.load_gather(xbuf, [q * FEAT + cat])
            gidx[pl.ds(r0, 16)] = xv.astype(jnp.int32) + cat * PV
            return 0

        lax.fori_loop(0, S16, idx_body, 0, unroll=4)

        handles = [
            pltpu.async_copy(
                tab_hbm.at[gidx.at[pl.ds(j * GB, GB)]],
                gbuf.at[pl.ds(j * GB, GB)],
                gsem,
            )
            for j in range(NG)
        ]

        pltpu.sync_copy(proj_hbm.at[pl.ds(tok0, T)], obuf)

        for h in handles:
            h.wait()

        def acc_body(t, _):
            r0 = t * NUM_CAT
            ve = jnp.zeros((16,), jnp.float32)
            vo = jnp.zeros((16,), jnp.float32)
            for i in range(NUM_CAT):
                xw = gbuf[r0 + i, pl.ds(0, 16)]
                a, b2 = plsc.unpack(
                    plsc.bitcast(xw, jnp.bfloat16),
                    format=plsc.PackFormat.INTERLEAVED,
                )
                ve = ve + a
                vo = vo + b2
            ts = jnp.zeros((16,), jnp.int32) + t
            pe = plsc.load_gather(obuf, [ts, ev])
            po = plsc.load_gather(obuf, [ts, od])
            plsc.store_scatter(obuf, [ts, ev], ve + pe)
            plsc.store_scatter(obuf, [ts, od], vo + po)
            return 0

        lax.fori_loop(0, T, acc_body, 0)

        pltpu.sync_copy(obuf, out_hbm.at[pl.ds(tok0, T)])
        return 0

    lax.fori_loop(0, CHUNKS, chunk_body, 0)


def kernel(x, tables, W, b):
    tt = jnp.transpose(tables, (0, 2, 1))
    tail = jnp.pad(tt[:, :, TAIL0:], ((0, 0), (0, 0), (0, TAIL0 + VB1 - ROWS)))
    packed = _pack_sc(tt, tail)
    tabf = packed.reshape(NPW16 // 16, 16)
    xf = x.reshape(N, FEAT)
    proj = _proj_tc(xf, W.T, b.reshape(1, EMB))
    out = _sc_lookup(xf.reshape(N * FEAT), tabf, proj)
    return out.reshape(B0, B1, EMB)
